# baseline (speedup 1.0000x reference)
import jax
import jax.numpy as jnp
from jax import lax
from jax.experimental import pallas as pl
from jax.experimental.pallas import tpu as pltpu
from jax.experimental.pallas import tpu_sc as plsc

B, F, V, D = 4096, 26, 100000, 128
DENSE, H = 13, 1024
KE = F * D
N = B * F

_NC, _NS = 2, 16
NW = _NC * _NS
PER_W = N // NW
CHUNK = 104
N_CH = PER_W // CHUNK
NBUF = 4

BM = 512


def _gather_body(idx_hbm, table_hbm, out_hbm, idx_v,
                 b0_v, b1_v, b2_v, b3_v,
                 g0, g1, g2, g3, w0, w1, w2, w3):
    wid = lax.axis_index("s") * _NC + lax.axis_index("c")
    base = wid * PER_W
    bufs = (b0_v, b1_v, b2_v, b3_v)
    gsems = (g0, g1, g2, g3)
    wsems = (w0, w1, w2, w3)

    def out_at(c):
        return out_hbm.at[pl.ds(pl.multiple_of(base + c * CHUNK, 8), CHUNK)]

    pltpu.sync_copy(idx_hbm.at[wid], idx_v)
    pltpu.async_copy(table_hbm.at[idx_v.at[0]], bufs[0], gsems[0])
    pltpu.async_copy(table_hbm.at[idx_v.at[1]], bufs[1], gsems[1])

    def body(i, carry):
        for b in range(NBUF):
            c = i * NBUF + b
            sp = (b + 2) % NBUF

            @pl.when(c >= 2)
            def _():
                pltpu.make_async_copy(bufs[sp], out_at(c - 2), wsems[sp]).wait()

            @pl.when(c + 2 < N_CH)
            def _():
                pltpu.async_copy(
                    table_hbm.at[idx_v.at[jnp.minimum(c + 2, N_CH - 1)]],
                    bufs[sp], gsems[sp])

            pltpu.make_async_copy(table_hbm.at[idx_v.at[c]],
                                  bufs[b], gsems[b]).wait()
            pltpu.async_copy(bufs[b], out_at(c), wsems[b])
        return carry

    lax.fori_loop(0, N_CH // NBUF, body, 0)
    pltpu.make_async_copy(bufs[(N_CH - 2) % NBUF], out_at(N_CH - 2),
                          wsems[(N_CH - 2) % NBUF]).wait()
    pltpu.make_async_copy(bufs[(N_CH - 1) % NBUF], out_at(N_CH - 1),
                          wsems[(N_CH - 1) % NBUF]).wait()


def _sc_gather(idx3, table):
    mesh = plsc.VectorSubcoreMesh(core_axis_name="c", subcore_axis_name="s")
    f = pl.kernel(
        _gather_body,
        out_type=jax.ShapeDtypeStruct((N, D), jnp.float32),
        mesh=mesh,
        scratch_types=(
            [pltpu.VMEM((N_CH, CHUNK), jnp.int32)]
            + [pltpu.VMEM((CHUNK, D), jnp.float32)] * NBUF
            + [pltpu.SemaphoreType.DMA] * (2 * NBUF)
        ),
    )
    return f(idx3, table)


def _towers_body(fc_ref, dn_ref, w1c_ref, w1v_ref,
                 b1c_ref, b1v_ref, w2c_ref, w2v_ref, b2_ref,
                 octr_ref, ocvr_ref):
    accs = []
    for w1 in (w1c_ref, w1v_ref):
        acc = None
        for f in range(F):
            xf = fc_ref[f].astype(jnp.bfloat16)
            wf = w1[pl.ds(f * D, D), :]
            d = jnp.dot(xf, wf, preferred_element_type=jnp.float32)
            acc = d if acc is None else acc + d
        xd = dn_ref[...].astype(jnp.bfloat16)
        wd = w1[pl.ds(KE, DENSE), :]
        acc = acc + jnp.dot(xd, wd, preferred_element_type=jnp.float32)
        accs.append(acc)
    for j, (acc, b1, w2, oref) in enumerate((
        (accs[0], b1c_ref, w2c_ref, octr_ref),
        (accs[1], b1v_ref, w2v_ref, ocvr_ref),
    )):
        h = jnp.maximum(acc + b1[...], 0.0)
        logit = jnp.sum(h * w2[...], axis=1, keepdims=True) + b2_ref[0, j]
        oref[...] = 1.0 / (1.0 + jnp.exp(-logit))


def _towers(fc3, dense, w1c, w1v, b1c, b1v, w2c, w2v, b2):
    nb = B // BM
    rep = lambda b: (0, 0)
    return pl.pallas_call(
        _towers_body,
        grid=(nb,),
        in_specs=[
            pl.BlockSpec((F, BM, D), lambda b: (0, b, 0)),
            pl.BlockSpec((BM, DENSE), lambda b: (b, 0)),
            pl.BlockSpec((KE + DENSE, H), rep),
            pl.BlockSpec((KE + DENSE, H), rep),
            pl.BlockSpec((1, H), rep),
            pl.BlockSpec((1, H), rep),
            pl.BlockSpec((1, H), rep),
            pl.BlockSpec((1, H), rep),
            pl.BlockSpec(memory_space=pltpu.SMEM),
        ],
        out_specs=[
            pl.BlockSpec((BM, 1), lambda b: (b, 0)),
            pl.BlockSpec((BM, 1), lambda b: (b, 0)),
        ],
        out_shape=[
            jax.ShapeDtypeStruct((B, 1), jnp.float32),
            jax.ShapeDtypeStruct((B, 1), jnp.float32),
        ],
    )(fc3, dense, w1c, w1v, b1c, b1v, w2c, w2v, b2)


def kernel(cat_fea_list, dense_features, table,
           W1_ctr, b1_ctr, W2_ctr, b2_ctr,
           W1_cvr, b1_cvr, W2_cvr, b2_cvr):
    idx3 = cat_fea_list.T.reshape(NW, N_CH, CHUNK)
    emb = _sc_gather(idx3, table)
    fc3 = emb.reshape(F, B, D)
    b2 = jnp.concatenate([b2_ctr, b2_cvr]).reshape(1, 2)
    octr, ocvr = _towers(fc3, dense_features,
                         W1_ctr.astype(jnp.bfloat16), W1_cvr.astype(jnp.bfloat16),
                         b1_ctr.reshape(1, H), b1_cvr.reshape(1, H),
                         W2_ctr.reshape(1, H), W2_cvr.reshape(1, H), b2)
    return octr[:, 0], ocvr[:, 0]


# speedup vs baseline: 1.4109x; 1.4109x over previous
import jax
import jax.numpy as jnp
from jax import lax
from jax.experimental import pallas as pl
from jax.experimental.pallas import tpu as pltpu
from jax.experimental.pallas import tpu_sc as plsc

B, F, V, D = 4096, 26, 100000, 128
DENSE, H = 13, 1024
KE = F * D
PAIRS = F // 2
NS = PAIRS * B

_NC, _NS = 2, 16
NW = _NC * _NS
PER_W = NS // NW
CHUNK = 104
N_CH = PER_W // CHUNK
NBUF = 4

BM = 512


def _gather_body(idxe_hbm, idxo_hbm, table_hbm, oute_hbm, outo_hbm,
                 idxe_v, idxo_v, b0_v, b1_v, b2_v, b3_v,
                 g0, g1, g2, g3, w0, w1, w2, w3):
    wid = lax.axis_index("s") * _NC + lax.axis_index("c")
    base = wid * PER_W
    bufs = (b0_v, b1_v, b2_v, b3_v)
    gsems = (g0, g1, g2, g3)
    wsems = (w0, w1, w2, w3)

    pltpu.sync_copy(idxe_hbm.at[wid], idxe_v)
    pltpu.sync_copy(idxo_hbm.at[wid], idxo_v)

    for idx_v, out_hbm in ((idxe_v, oute_hbm), (idxo_v, outo_hbm)):

        def out_at(c):
            return out_hbm.at[pl.ds(pl.multiple_of(base + c * CHUNK, 8), CHUNK)]

        pltpu.async_copy(table_hbm.at[idx_v.at[0]], bufs[0], gsems[0])
        pltpu.async_copy(table_hbm.at[idx_v.at[1]], bufs[1], gsems[1])

        def body(i, carry):
            for b in range(NBUF):
                c = i * NBUF + b
                sp = (b + 2) % NBUF

                @pl.when(c >= 2)
                def _():
                    pltpu.make_async_copy(bufs[sp], out_at(c - 2),
                                          wsems[sp]).wait()

                @pl.when(c + 2 < N_CH)
                def _():
                    pltpu.async_copy(
                        table_hbm.at[idx_v.at[jnp.minimum(c + 2, N_CH - 1)]],
                        bufs[sp], gsems[sp])

                pltpu.make_async_copy(table_hbm.at[idx_v.at[c]],
                                      bufs[b], gsems[b]).wait()
                pltpu.async_copy(bufs[b], out_at(c), wsems[b])
            return carry

        lax.fori_loop(0, N_CH // NBUF, body, 0)
        pltpu.make_async_copy(bufs[(N_CH - 2) % NBUF], out_at(N_CH - 2),
                              wsems[(N_CH - 2) % NBUF]).wait()
        pltpu.make_async_copy(bufs[(N_CH - 1) % NBUF], out_at(N_CH - 1),
                              wsems[(N_CH - 1) % NBUF]).wait()


def _sc_gather(idxe3, idxo3, table):
    mesh = plsc.VectorSubcoreMesh(core_axis_name="c", subcore_axis_name="s")
    f = pl.kernel(
        _gather_body,
        out_type=(jax.ShapeDtypeStruct((NS, D), jnp.float32),
                  jax.ShapeDtypeStruct((NS, D), jnp.float32)),
        mesh=mesh,
        scratch_types=(
            [pltpu.VMEM((N_CH, CHUNK), jnp.int32)] * 2
            + [pltpu.VMEM((CHUNK, D), jnp.float32)] * NBUF
            + [pltpu.SemaphoreType.DMA] * (2 * NBUF)
        ),
    )
    return f(idxe3, idxo3, table)


def _towers_body(fce_ref, fco_ref, dn_ref, w1c_ref, w1v_ref,
                 b1c_ref, b1v_ref, w2c_ref, w2v_ref, b2_ref,
                 octr_ref, ocvr_ref):
    xs = []
    for g in range(PAIRS):
        xe = fce_ref[g].astype(jnp.bfloat16)
        xo = fco_ref[g].astype(jnp.bfloat16)
        xs.append(jnp.concatenate([xe, xo], axis=1))
    xd = dn_ref[...].astype(jnp.bfloat16)
    accs = []
    for w1 in (w1c_ref, w1v_ref):
        acc = None
        for g in range(PAIRS):
            wg = w1[pl.ds(g * 2 * D, 2 * D), :].astype(jnp.bfloat16)
            d = jnp.dot(xs[g], wg, preferred_element_type=jnp.float32)
            acc = d if acc is None else acc + d
        wd = w1[pl.ds(KE, DENSE), :].astype(jnp.bfloat16)
        acc = acc + jnp.dot(xd, wd, preferred_element_type=jnp.float32)
        accs.append(acc)
    for j, (acc, b1, w2, oref) in enumerate((
        (accs[0], b1c_ref, w2c_ref, octr_ref),
        (accs[1], b1v_ref, w2v_ref, ocvr_ref),
    )):
        h = jnp.maximum(acc + b1[...], 0.0)
        logit = jnp.sum(h * w2[...], axis=1, keepdims=True) + b2_ref[0, j]
        oref[...] = 1.0 / (1.0 + jnp.exp(-logit))


def _towers(fce, fco, dense, w1c, w1v, b1c, b1v, w2c, w2v, b2):
    nb = B // BM
    rep = lambda b: (0, 0)
    return pl.pallas_call(
        _towers_body,
        grid=(nb,),
        in_specs=[
            pl.BlockSpec((PAIRS, BM, D), lambda b: (0, b, 0)),
            pl.BlockSpec((PAIRS, BM, D), lambda b: (0, b, 0)),
            pl.BlockSpec((BM, DENSE), lambda b: (b, 0)),
            pl.BlockSpec((KE + DENSE, H), rep),
            pl.BlockSpec((KE + DENSE, H), rep),
            pl.BlockSpec((1, H), rep),
            pl.BlockSpec((1, H), rep),
            pl.BlockSpec((1, H), rep),
            pl.BlockSpec((1, H), rep),
            pl.BlockSpec(memory_space=pltpu.SMEM),
        ],
        out_specs=[
            pl.BlockSpec((BM, 1), lambda b: (b, 0)),
            pl.BlockSpec((BM, 1), lambda b: (b, 0)),
        ],
        out_shape=[
            jax.ShapeDtypeStruct((B, 1), jnp.float32),
            jax.ShapeDtypeStruct((B, 1), jnp.float32),
        ],
    )(fce, fco, dense, w1c, w1v, b1c, b1v, w2c, w2v, b2)


def kernel(cat_fea_list, dense_features, table,
           W1_ctr, b1_ctr, W2_ctr, b2_ctr,
           W1_cvr, b1_cvr, W2_cvr, b2_cvr):
    catT = cat_fea_list.T
    idxe3 = catT[0::2].reshape(NW, N_CH, CHUNK)
    idxo3 = catT[1::2].reshape(NW, N_CH, CHUNK)
    embe, embo = _sc_gather(idxe3, idxo3, table)
    fce = embe.reshape(PAIRS, B, D)
    fco = embo.reshape(PAIRS, B, D)
    b2 = jnp.concatenate([b2_ctr, b2_cvr]).reshape(1, 2)
    octr, ocvr = _towers(fce, fco, dense_features, W1_ctr, W1_cvr,
                         b1_ctr.reshape(1, H), b1_cvr.reshape(1, H),
                         W2_ctr.reshape(1, H), W2_cvr.reshape(1, H), b2)
    return octr[:, 0], ocvr[:, 0]


# speedup vs baseline: 1.4672x; 1.0399x over previous
import jax
import jax.numpy as jnp
from jax import lax
from jax.experimental import pallas as pl
from jax.experimental.pallas import tpu as pltpu
from jax.experimental.pallas import tpu_sc as plsc

B, F, V, D = 4096, 26, 100000, 128
DENSE, H = 13, 1024
KE = F * D
PAIRS = F // 2
NS = PAIRS * B

_NC, _NS = 2, 16
NW = _NC * _NS
PER_W = NS // NW
CHUNK = 104
N_CH = PER_W // CHUNK
NBUF = 4

BM = 512


def _gather_body(idx_hbm, table_hbm, oute_hbm, outo_hbm,
                 idxe_v, idxo_v, b0_v, b1_v, b2_v, b3_v,
                 g0, g1, g2, g3, w0, w1, w2, w3):
    wid = lax.axis_index("s") * _NC + lax.axis_index("c")
    base = wid * PER_W
    bufs = (b0_v, b1_v, b2_v, b3_v)
    gsems = (g0, g1, g2, g3)
    wsems = (w0, w1, w2, w3)

    pltpu.sync_copy(idx_hbm.at[0, wid], idxe_v)
    pltpu.sync_copy(idx_hbm.at[1, wid], idxo_v)

    for idx_v, out_hbm in ((idxe_v, oute_hbm), (idxo_v, outo_hbm)):

        def out_at(c):
            return out_hbm.at[pl.ds(pl.multiple_of(base + c * CHUNK, 8), CHUNK)]

        pltpu.async_copy(table_hbm.at[idx_v.at[0]], bufs[0], gsems[0])
        pltpu.async_copy(table_hbm.at[idx_v.at[1]], bufs[1], gsems[1])

        def body(i, carry):
            for b in range(NBUF):
                c = i * NBUF + b
                sp = (b + 2) % NBUF

                @pl.when(c >= 2)
                def _():
                    pltpu.make_async_copy(bufs[sp], out_at(c - 2),
                                          wsems[sp]).wait()

                @pl.when(c + 2 < N_CH)
                def _():
                    pltpu.async_copy(
                        table_hbm.at[idx_v.at[jnp.minimum(c + 2, N_CH - 1)]],
                        bufs[sp], gsems[sp])

                pltpu.make_async_copy(table_hbm.at[idx_v.at[c]],
                                      bufs[b], gsems[b]).wait()
                pltpu.async_copy(bufs[b], out_at(c), wsems[b])
            return carry

        lax.fori_loop(0, N_CH // NBUF, body, 0)
        pltpu.make_async_copy(bufs[(N_CH - 2) % NBUF], out_at(N_CH - 2),
                              wsems[(N_CH - 2) % NBUF]).wait()
        pltpu.make_async_copy(bufs[(N_CH - 1) % NBUF], out_at(N_CH - 1),
                              wsems[(N_CH - 1) % NBUF]).wait()


def _sc_gather(idxeo, table):
    mesh = plsc.VectorSubcoreMesh(core_axis_name="c", subcore_axis_name="s")
    f = pl.kernel(
        _gather_body,
        out_type=(jax.ShapeDtypeStruct((NS, D), jnp.float32),
                  jax.ShapeDtypeStruct((NS, D), jnp.float32)),
        mesh=mesh,
        scratch_types=(
            [pltpu.VMEM((N_CH, CHUNK), jnp.int32)] * 2
            + [pltpu.VMEM((CHUNK, D), jnp.float32)] * NBUF
            + [pltpu.SemaphoreType.DMA] * (2 * NBUF)
        ),
    )
    return f(idxeo, table)


def _towers_body(fce_ref, fco_ref, dn_ref, w1c_ref, w1v_ref,
                 b1c_ref, b1v_ref, w2c_ref, w2v_ref, b2c_ref, b2v_ref,
                 octr_ref, ocvr_ref):
    xs = []
    for g in range(PAIRS):
        xe = fce_ref[g].astype(jnp.bfloat16)
        xo = fco_ref[g].astype(jnp.bfloat16)
        xs.append(jnp.concatenate([xe, xo], axis=1))
    xd = dn_ref[...].astype(jnp.bfloat16)
    accs = []
    for w1 in (w1c_ref, w1v_ref):
        acc = None
        for g in range(PAIRS):
            wg = w1[pl.ds(g * 2 * D, 2 * D), :].astype(jnp.bfloat16)
            d = jnp.dot(xs[g], wg, preferred_element_type=jnp.float32)
            acc = d if acc is None else acc + d
        wd = w1[pl.ds(KE, DENSE), :].astype(jnp.bfloat16)
        acc = acc + jnp.dot(xd, wd, preferred_element_type=jnp.float32)
        accs.append(acc)
    for acc, b1, w2, b2, oref in (
        (accs[0], b1c_ref, w2c_ref, b2c_ref, octr_ref),
        (accs[1], b1v_ref, w2v_ref, b2v_ref, ocvr_ref),
    ):
        h = jnp.maximum(acc + b1[...], 0.0)
        logit = jnp.sum(h * w2[...], axis=1) + b2[0]
        oref[...] = 1.0 / (1.0 + jnp.exp(-logit))


def _towers(fce, fco, dense, w1c, w1v, b1c, b1v, w2c, w2v, b2c, b2v):
    nb = B // BM
    rep = lambda b: (0, 0)
    rep1 = lambda b: (0,)
    return pl.pallas_call(
        _towers_body,
        grid=(nb,),
        in_specs=[
            pl.BlockSpec((PAIRS, BM, D), lambda b: (0, b, 0)),
            pl.BlockSpec((PAIRS, BM, D), lambda b: (0, b, 0)),
            pl.BlockSpec((BM, DENSE), lambda b: (b, 0)),
            pl.BlockSpec((KE + DENSE, H), rep),
            pl.BlockSpec((KE + DENSE, H), rep),
            pl.BlockSpec((H,), rep1),
            pl.BlockSpec((H,), rep1),
            pl.BlockSpec((H,), rep1),
            pl.BlockSpec((H,), rep1),
            pl.BlockSpec(memory_space=pltpu.SMEM),
            pl.BlockSpec(memory_space=pltpu.SMEM),
        ],
        out_specs=[
            pl.BlockSpec((BM,), lambda b: (b,)),
            pl.BlockSpec((BM,), lambda b: (b,)),
        ],
        out_shape=[
            jax.ShapeDtypeStruct((B,), jnp.float32),
            jax.ShapeDtypeStruct((B,), jnp.float32),
        ],
    )(fce, fco, dense, w1c, w1v, b1c, b1v, w2c, w2v, b2c, b2v)


def kernel(cat_fea_list, dense_features, table,
           W1_ctr, b1_ctr, W2_ctr, b2_ctr,
           W1_cvr, b1_cvr, W2_cvr, b2_cvr):
    idxeo = cat_fea_list.reshape(B, PAIRS, 2).transpose(2, 1, 0)
    idxeo = idxeo.reshape(2, NW, N_CH, CHUNK)
    embe, embo = _sc_gather(idxeo, table)
    fce = embe.reshape(PAIRS, B, D)
    fco = embo.reshape(PAIRS, B, D)
    octr, ocvr = _towers(fce, fco, dense_features, W1_ctr, W1_cvr,
                         b1_ctr, b1_cvr,
                         W2_ctr.reshape(H), W2_cvr.reshape(H),
                         b2_ctr, b2_cvr)
    return octr, ocvr
